# Initial kernel scaffold; baseline (speedup 1.0000x reference)
import jax
import jax.numpy as jnp
from jax.experimental import pallas as pl


def kernel(x, embed_matrix):
    raise NotImplementedError("write your pallas kernel here")


# speedup vs baseline: 109.9472x; 109.9472x over previous
import jax
import jax.numpy as jnp
from jax import lax
from jax.experimental import pallas as pl
from jax.experimental.pallas import tpu as pltpu
from jax.experimental.pallas import tpu_sc as plsc

B = 16384
I = 100
IP = 112
V = 31
E = 64
NC = 2
NS = 16
EH = E // NC
TROWS = I * V + 4
RPT = B // NS
RC = 64
NCHUNK = RPT // RC


def _sc_kernel(x_hbm, tab_hbm, out_hbm, tab_v, x_v, out_v):
    c = lax.axis_index("c")
    s = lax.axis_index("s")

    pltpu.sync_copy(tab_hbm.at[c], tab_v)

    def chunk_body(k, _):
        base = s * RPT + k * RC
        pltpu.sync_copy(x_hbm.at[pl.ds(base, RC), :], x_v)

        def row_body(r, _):
            a0 = jnp.zeros((16,), jnp.float32)
            a1 = jnp.zeros((16,), jnp.float32)
            for i16 in range(IP // 16):
                vidx = x_v[r, pl.ds(i16 * 16, 16)]
                for l in range(16):
                    j = vidx[l]
                    a0 = a0 + tab_v[j, pl.ds(0, 16)]
                    a1 = a1 + tab_v[j, pl.ds(16, 16)]
            out_v[r, pl.ds(0, 16)] = a0
            out_v[r, pl.ds(16, 16)] = a1
            return 0

        lax.fori_loop(0, RC, row_body, 0)
        pltpu.sync_copy(out_v, out_hbm.at[c, pl.ds(base, RC), :])
        return 0

    lax.fori_loop(0, NCHUNK, chunk_body, 0)


@jax.jit
def _run(x_flat, tab2):
    mesh = plsc.VectorSubcoreMesh(core_axis_name="c", subcore_axis_name="s")
    f = pl.kernel(
        _sc_kernel,
        out_type=jax.ShapeDtypeStruct((NC, B, EH), jnp.float32),
        mesh=mesh,
        scratch_types=[
            pltpu.VMEM((TROWS, EH), jnp.float32),
            pltpu.VMEM((RC, IP), jnp.int32),
            pltpu.VMEM((RC, EH), jnp.float32),
        ],
        compiler_params=pltpu.CompilerParams(use_tc_tiling_on_sc=False),
    )
    return f(x_flat, tab2)


def kernel(x, embed_matrix):
    x = x.astype(jnp.int32)
    x_flat = x + (jnp.arange(I, dtype=jnp.int32) * V)[None, :]
    x_flat = jnp.concatenate(
        [x_flat, jnp.full((B, IP - I), I * V, jnp.int32)], axis=1
    )
    flat = embed_matrix.reshape(I * V, E)
    flat = jnp.concatenate(
        [flat, jnp.zeros((TROWS - I * V, E), jnp.float32)], axis=0
    )
    tab2 = flat.reshape(TROWS, NC, EH).transpose(1, 0, 2)
    out3 = _run(x_flat, tab2)
    return out3.transpose(1, 0, 2).reshape(B, E)


# speedup vs baseline: 120.4862x; 1.0959x over previous
import jax
import jax.numpy as jnp
import numpy as np
from jax import lax
from jax.experimental import pallas as pl
from jax.experimental.pallas import tpu as pltpu
from jax.experimental.pallas import tpu_sc as plsc

B = 16384
I = 100
IP = 112
V = 31
E = 64
NC = 2
NS = 16
EH = E // NC
TROWS = I * V + 4
RPT = B // NS
RC = 64
NCHUNK = RPT // RC

_HIMASK = np.int32(-65536)


def _sc_kernel(x_hbm, tab_hbm, out_hbm, tab_v, x_v, out_v):
    c = lax.axis_index("c")
    s = lax.axis_index("s")

    pltpu.sync_copy(tab_hbm.at[c], tab_v)

    def chunk_body(k, _):
        base = s * RPT + k * RC
        pltpu.sync_copy(x_hbm.at[pl.ds(base, RC), :], x_v)

        def row_body(r, _):
            z = jnp.zeros((16,), jnp.float32)
            acc = [z, z, z, z]
            for i16 in range(IP // 16):
                vidx = x_v[r, pl.ds(i16 * 16, 16)]
                for l in range(16):
                    j = vidx[l]
                    w = tab_v[j]
                    lo = plsc.bitcast(lax.shift_left(w, 16), jnp.float32)
                    hi = plsc.bitcast(lax.bitwise_and(w, _HIMASK),
                                      jnp.float32)
                    p = l & 1
                    acc[p] = acc[p] + lo
                    acc[2 + p] = acc[2 + p] + hi
            out_v[r, pl.ds(0, 16)] = acc[0] + acc[1]
            out_v[r, pl.ds(16, 16)] = acc[2] + acc[3]
            return 0

        lax.fori_loop(0, RC, row_body, 0)
        pltpu.sync_copy(out_v, out_hbm.at[c, pl.ds(base, RC), :])
        return 0

    lax.fori_loop(0, NCHUNK, chunk_body, 0)


@jax.jit
def _run(x_flat, tab2):
    mesh = plsc.VectorSubcoreMesh(core_axis_name="c", subcore_axis_name="s")
    f = pl.kernel(
        _sc_kernel,
        out_type=jax.ShapeDtypeStruct((NC, B, EH), jnp.float32),
        mesh=mesh,
        scratch_types=[
            pltpu.VMEM((TROWS, EH // 2), jnp.int32),
            pltpu.VMEM((RC, IP), jnp.int32),
            pltpu.VMEM((RC, EH), jnp.float32),
        ],
        compiler_params=pltpu.CompilerParams(
            use_tc_tiling_on_sc=False, needs_layout_passes=False
        ),
    )
    return f(x_flat, tab2)


_PERM = np.empty((EH,), np.int32)
_PERM[0::2] = np.arange(16)
_PERM[1::2] = np.arange(16) + 16


def kernel(x, embed_matrix):
    x = x.astype(jnp.int32)
    x_flat = x + (jnp.arange(I, dtype=jnp.int32) * V)[None, :]
    x_flat = jnp.concatenate(
        [x_flat, jnp.full((B, IP - I), I * V, jnp.int32)], axis=1
    )
    flat = embed_matrix.reshape(I * V, E)
    flat = jnp.concatenate(
        [flat, jnp.zeros((TROWS - I * V, E), jnp.float32)], axis=0
    )
    tab2 = flat.reshape(TROWS, NC, EH).transpose(1, 0, 2)
    tab2 = tab2[:, :, _PERM].astype(jnp.bfloat16)
    tab2 = lax.bitcast_convert_type(
        tab2.reshape(NC, TROWS, EH // 2, 2), jnp.int32
    )
    out3 = _run(x_flat, tab2)
    return out3.transpose(1, 0, 2).reshape(B, E)


# speedup vs baseline: 139.3364x; 1.1565x over previous
import jax
import jax.numpy as jnp
import numpy as np
from jax import lax
from jax.experimental import pallas as pl
from jax.experimental.pallas import tpu as pltpu
from jax.experimental.pallas import tpu_sc as plsc

B = 16384
I = 100
IP = 112
V = 31
E = 64
NC = 2
NS = 16
EH = E // NC
TROWS = I * V + 4
RPT = B // NS
RC = 64
NCHUNK = RPT // RC

_HIMASK = np.int32(-65536)


def _sc_kernel(x_hbm, tab_hbm, out_hbm, tab_v, x_v, out_v):
    c = lax.axis_index("c")
    s = lax.axis_index("s")

    pltpu.sync_copy(tab_hbm.at[c], tab_v)

    def chunk_body(k, _):
        base = s * RPT + k * RC
        pltpu.sync_copy(x_hbm.at[pl.ds(base, RC), :], x_v)

        def row_body(r, _):
            z = jnp.zeros((16,), jnp.float32)
            acc = [z, z, z, z]
            for i16 in range(IP // 16):
                vidx = x_v[r, pl.ds(i16 * 16, 16)]
                for l in range(0, 16, 2):
                    ps = tab_v[vidx[l]] + tab_v[vidx[l + 1]]
                    w = plsc.bitcast(ps, jnp.int32)
                    lo = plsc.bitcast(lax.shift_left(w, 16), jnp.float32)
                    hi = plsc.bitcast(lax.bitwise_and(w, _HIMASK),
                                      jnp.float32)
                    p = (l >> 1) & 1
                    acc[p] = acc[p] + lo
                    acc[2 + p] = acc[2 + p] + hi
            out_v[r, pl.ds(0, 16)] = acc[0] + acc[1]
            out_v[r, pl.ds(16, 16)] = acc[2] + acc[3]
            return 0

        lax.fori_loop(0, RC, row_body, 0)
        pltpu.sync_copy(out_v, out_hbm.at[c, pl.ds(base, RC), :])
        return 0

    lax.fori_loop(0, NCHUNK, chunk_body, 0)


@jax.jit
def _run(x_flat, tab2):
    mesh = plsc.VectorSubcoreMesh(core_axis_name="c", subcore_axis_name="s")
    f = pl.kernel(
        _sc_kernel,
        out_type=jax.ShapeDtypeStruct((NC, B, EH), jnp.float32),
        mesh=mesh,
        scratch_types=[
            pltpu.VMEM((TROWS, EH), jnp.bfloat16),
            pltpu.VMEM((RC, IP), jnp.int32),
            pltpu.VMEM((RC, EH), jnp.float32),
        ],
        compiler_params=pltpu.CompilerParams(
            use_tc_tiling_on_sc=False, needs_layout_passes=False
        ),
    )
    return f(x_flat, tab2)


_PERM = np.empty((EH,), np.int32)
_PERM[0::2] = np.arange(16)
_PERM[1::2] = np.arange(16) + 16


def kernel(x, embed_matrix):
    x = x.astype(jnp.int32)
    x_flat = x + (jnp.arange(I, dtype=jnp.int32) * V)[None, :]
    x_flat = jnp.concatenate(
        [x_flat, jnp.full((B, IP - I), I * V, jnp.int32)], axis=1
    )
    flat = embed_matrix.reshape(I * V, E)
    flat = jnp.concatenate(
        [flat, jnp.zeros((TROWS - I * V, E), jnp.float32)], axis=0
    )
    tab2 = flat.reshape(TROWS, NC, EH).transpose(1, 0, 2)
    tab2 = tab2[:, :, _PERM].astype(jnp.bfloat16)
    out3 = _run(x_flat, tab2)
    return out3.transpose(1, 0, 2).reshape(B, E)
